# baseline (speedup 1.0000x reference)
import functools

import jax
import jax.numpy as jnp
from jax import lax
from jax.experimental import pallas as pl
from jax.experimental.pallas import tpu as pltpu
from jax.experimental.pallas import tpu_sc as plsc

B = 16384
IN_DIM = 52
N_EMB = 26
VOCAB = 100000
EMB = 16
OUT_DIM = N_EMB * EMB + N_EMB
NROW = N_EMB * VOCAB

NC = 2
NS = 16
L = 16
NW = NC * NS
RPW = B // NW
NCHUNK = RPW // L
IDXW = 128
NIDX = RPW // IDXW
PERW = RPW * N_EMB
HRPW = RPW // 2
HPERW = HRPW * N_EMB

VT = VOCAB // 128
UNITS = N_EMB * VT
PADK = 636
TAILV = VOCAB - VT * 128


def _tp_body(x_hbm, tabt_hbm, packed_hbm, idxf_hbm, ptf_hbm,
             xs_v, idxs_v, pts_v, stage, p_v, st_t, p_t,
             sin0, sin1, sout0, sout1):
    w = lax.axis_index("s") * NC + lax.axis_index("c")
    base = w * RPW

    iota = lax.iota(jnp.int32, L)

    def half_body(cb, c):
        pltpu.sync_copy(x_hbm.at[:, pl.ds(base + cb * HRPW, HRPW)], xs_v)

        def col_body(i, c2):
            def idx_chunk(k, cc):
                vals = xs_v[i, pl.ds(k * L, L)] + i * VOCAB
                idxs_v[pl.ds(i * HRPW + k * L, L)] = vals
                return cc
            lax.fori_loop(0, HRPW // L, idx_chunk, 0)
            return c2
        lax.fori_loop(0, N_EMB, col_body, 0)

        def pt_body(j, c2):
            def pt_chunk(k, cc):
                vals = xs_v[N_EMB + j, pl.ds(k * L, L)].astype(jnp.float32)
                pos = (k * L + iota) * N_EMB + j
                plsc.store_scatter(pts_v, [pos], vals)
                return cc
            lax.fori_loop(0, HRPW // L, pt_chunk, 0)
            return c2
        lax.fori_loop(0, N_EMB, pt_body, 0)

        pltpu.sync_copy(idxs_v, idxf_hbm.at[pl.ds(w * PERW + cb * HPERW, HPERW)])
        pltpu.sync_copy(pts_v, ptf_hbm.at[pl.ds(w * PERW + cb * HPERW, HPERW)])
        return c
    lax.fori_loop(0, 2, half_body, 0)

    sins = (sin0, sin1)
    souts = (sout0, sout1)

    def unit_coords(k):
        u = w + k * NW
        u = lax.select(u < UNITS, u, w)
        i = u // VT
        vt = u % VT
        return i * 16, vt * 128, i * VOCAB + vt * 128

    def issue_load(k, b):
        f0, v0, _ = unit_coords(k)
        pltpu.async_copy(
            tabt_hbm.at[pl.ds(f0, 16), pl.ds(v0, 128)], stage.at[b], sins[b]
        )

    def shuffle(b):
        for j in range(128):
            vals = plsc.load_gather(
                stage,
                [jnp.full((L,), b, jnp.int32), iota, jnp.full((L,), j, jnp.int32)],
            )
            p_v[b, pl.ds(j * L, L)] = vals

    issue_load(0, 0)
    issue_load(1, 1)

    def pipe_body(kk, c):
        for b in range(2):
            k = kk * 2 + b
            pltpu.make_async_copy(
                tabt_hbm.at[pl.ds(0, 16), pl.ds(0, 128)], stage.at[b], sins[b]
            ).wait()

            @pl.when(kk > 0)
            def _():
                pltpu.make_async_copy(
                    p_v.at[b], packed_hbm.at[pl.ds(0, 128 * EMB)], souts[b]
                ).wait()

            shuffle(b)
            _, _, q0 = unit_coords(k)
            pltpu.async_copy(
                p_v.at[b], packed_hbm.at[pl.ds(q0 * EMB, 128 * EMB)], souts[b]
            )

            @pl.when(k < PADK - 2)
            def _():
                issue_load(k + 2, b)
        return c

    lax.fori_loop(0, PADK // 2, pipe_body, 0)
    for b in range(2):
        pltpu.make_async_copy(
            p_v.at[b], packed_hbm.at[pl.ds(0, 128 * EMB)], souts[b]
        ).wait()

    @pl.when(w < N_EMB)
    def _():
        pltpu.sync_copy(
            tabt_hbm.at[pl.ds(w * 16, 16), pl.ds(VT * 128, TAILV)], st_t
        )
        for j in range(TAILV):
            vals = plsc.load_gather(
                st_t, [iota, jnp.full((L,), j, jnp.int32)]
            )
            p_t[pl.ds(j * L, L)] = vals
        pltpu.sync_copy(
            p_t, packed_hbm.at[pl.ds((w * VOCAB + VT * 128) * EMB, TAILV * EMB)]
        )


def _gt_body(idxf_hbm, ptf_hbm, packed_hbm, out_hbm, idx_v, g_v, f_v, pt1_v, sem):
    w = lax.axis_index("s") * NC + lax.axis_index("c")
    base = w * RPW

    pltpu.sync_copy(ptf_hbm.at[pl.ds(w * PERW, PERW)], pt1_v)

    def pt_body(r, c):
        f_v[r, pl.ds(0, L)] = pt1_v[pl.ds(r * N_EMB, L)]
        f_v[r, pl.ds(10, L)] = pt1_v[pl.ds(r * N_EMB + 10, L)]
        return c
    lax.fori_loop(0, RPW, pt_body, 0)
    pltpu.sync_copy(f_v, out_hbm.at[pl.ds(base, RPW), pl.ds(N_EMB * EMB, N_EMB)])

    def col_body(i, carry):
        pltpu.sync_copy(
            idxf_hbm.at[pl.ds(w * PERW + i * HRPW, HRPW)],
            idx_v.at[pl.ds(0, HRPW)],
        )
        pltpu.sync_copy(
            idxf_hbm.at[pl.ds(w * PERW + HPERW + i * HRPW, HRPW)],
            idx_v.at[pl.ds(HRPW, HRPW)],
        )
        cps = [
            pltpu.async_copy(
                packed_hbm.at[idx_v.at[pl.ds(j * IDXW, IDXW)]],
                g_v.at[pl.ds(j * IDXW, IDXW), :],
                sem,
            )
            for j in range(NIDX)
        ]
        for cp in cps:
            cp.wait()
        pltpu.sync_copy(g_v, out_hbm.at[pl.ds(base, RPW), pl.ds(i * EMB, EMB)])
        return carry

    lax.fori_loop(0, N_EMB, col_body, 0)


_mesh = plsc.VectorSubcoreMesh(core_axis_name="c", subcore_axis_name="s")

_repack = functools.partial(
    pl.kernel,
    mesh=_mesh,
    out_type=(
        jax.ShapeDtypeStruct((NROW * EMB,), jnp.float32),
        jax.ShapeDtypeStruct((B * N_EMB,), jnp.int32),
        jax.ShapeDtypeStruct((B * N_EMB,), jnp.float32),
    ),
    compiler_params=pltpu.CompilerParams(needs_layout_passes=False),
    scratch_types=[
        pltpu.VMEM((IN_DIM, HRPW), jnp.int32),
        pltpu.VMEM((HPERW,), jnp.int32),
        pltpu.VMEM((HPERW,), jnp.float32),
        pltpu.VMEM((2, 16, 128), jnp.float32),
        pltpu.VMEM((2, 128 * EMB), jnp.float32),
        pltpu.VMEM((16, TAILV), jnp.float32),
        pltpu.VMEM((TAILV * EMB,), jnp.float32),
        pltpu.SemaphoreType.DMA,
        pltpu.SemaphoreType.DMA,
        pltpu.SemaphoreType.DMA,
        pltpu.SemaphoreType.DMA,
    ],
)(_tp_body)

_gather = functools.partial(
    pl.kernel,
    mesh=_mesh,
    out_type=jax.ShapeDtypeStruct((B, OUT_DIM), jnp.float32),
    compiler_params=pltpu.CompilerParams(
        use_tc_tiling_on_sc=False, needs_layout_passes=False
    ),
    scratch_types=[
        pltpu.VMEM((RPW,), jnp.int32),
        pltpu.VMEM((RPW, EMB), jnp.float32),
        pltpu.VMEM((RPW, N_EMB), jnp.float32),
        pltpu.VMEM((PERW,), jnp.float32),
        pltpu.SemaphoreType.DMA,
    ],
)(_gt_body)


def kernel(x, emb_tables):
    tabt = jnp.transpose(emb_tables, (0, 2, 1)).reshape(N_EMB * EMB, VOCAB)
    packed, idxf, ptf = _repack(x.T, tabt)
    return _gather(idxf, ptf, packed.reshape(NROW, EMB))


# speedup vs baseline: 1.6182x; 1.6182x over previous
import functools

import jax
import jax.numpy as jnp
from jax import lax
from jax.experimental import pallas as pl
from jax.experimental.pallas import tpu as pltpu
from jax.experimental.pallas import tpu_sc as plsc

B = 16384
IN_DIM = 52
N_EMB = 26
VOCAB = 100000
EMB = 16
OUT_DIM = N_EMB * EMB + N_EMB
NROW = N_EMB * VOCAB

NC = 2
NS = 16
L = 16
NW = NC * NS
RPW = B // NW
NCHUNK = RPW // L
IDXW = 128
NIDX = RPW // IDXW
PERW = RPW * N_EMB
HRPW = RPW // 2
HPERW = HRPW * N_EMB

VT = VOCAB // 128
UNITS = N_EMB * VT
PADK = 636
TAILV = VOCAB - VT * 128


def _tp_body(x_hbm, tabt_hbm, packed_hbm, idxf_hbm, ptf_hbm,
             xs_v, idxs_v, pts_v, stage, p_v, st_t, p_t,
             sin0, sin1, sout0, sout1):
    w = lax.axis_index("s") * NC + lax.axis_index("c")
    base = w * RPW

    iota = lax.iota(jnp.int32, L)

    def half_body(cb, c):
        pltpu.sync_copy(x_hbm.at[:, pl.ds(base + cb * HRPW, HRPW)], xs_v)

        @plsc.parallel_loop(0, N_EMB * (HRPW // L), unroll=4)
        def _(ik):
            i = ik // (HRPW // L)
            k = ik % (HRPW // L)
            vals = xs_v[i, pl.ds(k * L, L)] + i * VOCAB
            idxs_v[pl.ds(i * HRPW + k * L, L)] = vals

        @plsc.parallel_loop(0, N_EMB * (HRPW // L), unroll=4)
        def _(jk):
            j = jk // (HRPW // L)
            k = jk % (HRPW // L)
            vals = xs_v[N_EMB + j, pl.ds(k * L, L)].astype(jnp.float32)
            pos = (k * L + iota) * N_EMB + j
            plsc.store_scatter(pts_v, [pos], vals)

        pltpu.sync_copy(idxs_v, idxf_hbm.at[pl.ds(w * PERW + cb * HPERW, HPERW)])
        pltpu.sync_copy(pts_v, ptf_hbm.at[pl.ds(w * PERW + cb * HPERW, HPERW)])
        return c
    lax.fori_loop(0, 2, half_body, 0)

    sins = (sin0, sin1)
    souts = (sout0, sout1)

    def unit_coords(k):
        u = w + k * NW
        u = lax.select(u < UNITS, u, w)
        i = u // VT
        vt = u % VT
        return i * 16, vt * 128, i * VOCAB + vt * 128

    def issue_load(k, b):
        f0, v0, _ = unit_coords(k)
        pltpu.async_copy(
            tabt_hbm.at[pl.ds(f0, 16), pl.ds(v0, 128)], stage.at[b], sins[b]
        )

    def shuffle(b):
        @plsc.parallel_loop(0, 128, unroll=8)
        def _(j):
            vals = plsc.load_gather(
                stage,
                [jnp.full((L,), b, jnp.int32), iota, jnp.full((L,), j, jnp.int32)],
            )
            p_v[b, pl.ds(j * L, L)] = vals

    issue_load(0, 0)
    issue_load(1, 1)

    def pipe_body(kk, c):
        for b in range(2):
            k = kk * 2 + b
            pltpu.make_async_copy(
                tabt_hbm.at[pl.ds(0, 16), pl.ds(0, 128)], stage.at[b], sins[b]
            ).wait()

            @pl.when(kk > 0)
            def _():
                pltpu.make_async_copy(
                    p_v.at[b], packed_hbm.at[pl.ds(0, 128 * EMB)], souts[b]
                ).wait()

            shuffle(b)
            _, _, q0 = unit_coords(k)
            pltpu.async_copy(
                p_v.at[b], packed_hbm.at[pl.ds(q0 * EMB, 128 * EMB)], souts[b]
            )

            @pl.when(k < PADK - 2)
            def _():
                issue_load(k + 2, b)
        return c

    lax.fori_loop(0, PADK // 2, pipe_body, 0)
    for b in range(2):
        pltpu.make_async_copy(
            p_v.at[b], packed_hbm.at[pl.ds(0, 128 * EMB)], souts[b]
        ).wait()

    @pl.when(w < N_EMB)
    def _():
        pltpu.sync_copy(
            tabt_hbm.at[pl.ds(w * 16, 16), pl.ds(VT * 128, TAILV)], st_t
        )
        @plsc.parallel_loop(0, TAILV, unroll=4)
        def _(j):
            vals = plsc.load_gather(
                st_t, [iota, jnp.full((L,), j, jnp.int32)]
            )
            p_t[pl.ds(j * L, L)] = vals
        pltpu.sync_copy(
            p_t, packed_hbm.at[pl.ds((w * VOCAB + VT * 128) * EMB, TAILV * EMB)]
        )


def _gt_body(idxf_hbm, ptf_hbm, packed_hbm, out_hbm, idx_v, g_v, f_v, pt1_v, sem):
    w = lax.axis_index("s") * NC + lax.axis_index("c")
    base = w * RPW

    pltpu.sync_copy(ptf_hbm.at[pl.ds(w * PERW, PERW)], pt1_v)

    @plsc.parallel_loop(0, RPW, unroll=4)
    def _(r):
        f_v[r, pl.ds(0, L)] = pt1_v[pl.ds(r * N_EMB, L)]
        f_v[r, pl.ds(10, L)] = pt1_v[pl.ds(r * N_EMB + 10, L)]
    pltpu.sync_copy(f_v, out_hbm.at[pl.ds(base, RPW), pl.ds(N_EMB * EMB, N_EMB)])

    def col_body(i, carry):
        pltpu.sync_copy(
            idxf_hbm.at[pl.ds(w * PERW + i * HRPW, HRPW)],
            idx_v.at[pl.ds(0, HRPW)],
        )
        pltpu.sync_copy(
            idxf_hbm.at[pl.ds(w * PERW + HPERW + i * HRPW, HRPW)],
            idx_v.at[pl.ds(HRPW, HRPW)],
        )
        cps = [
            pltpu.async_copy(
                packed_hbm.at[idx_v.at[pl.ds(j * IDXW, IDXW)]],
                g_v.at[pl.ds(j * IDXW, IDXW), :],
                sem,
            )
            for j in range(NIDX)
        ]
        for cp in cps:
            cp.wait()
        pltpu.sync_copy(g_v, out_hbm.at[pl.ds(base, RPW), pl.ds(i * EMB, EMB)])
        return carry

    lax.fori_loop(0, N_EMB, col_body, 0)


_mesh = plsc.VectorSubcoreMesh(core_axis_name="c", subcore_axis_name="s")

_repack = functools.partial(
    pl.kernel,
    mesh=_mesh,
    out_type=(
        jax.ShapeDtypeStruct((NROW * EMB,), jnp.float32),
        jax.ShapeDtypeStruct((B * N_EMB,), jnp.int32),
        jax.ShapeDtypeStruct((B * N_EMB,), jnp.float32),
    ),
    compiler_params=pltpu.CompilerParams(needs_layout_passes=False),
    scratch_types=[
        pltpu.VMEM((IN_DIM, HRPW), jnp.int32),
        pltpu.VMEM((HPERW,), jnp.int32),
        pltpu.VMEM((HPERW,), jnp.float32),
        pltpu.VMEM((2, 16, 128), jnp.float32),
        pltpu.VMEM((2, 128 * EMB), jnp.float32),
        pltpu.VMEM((16, TAILV), jnp.float32),
        pltpu.VMEM((TAILV * EMB,), jnp.float32),
        pltpu.SemaphoreType.DMA,
        pltpu.SemaphoreType.DMA,
        pltpu.SemaphoreType.DMA,
        pltpu.SemaphoreType.DMA,
    ],
)(_tp_body)

_gather = functools.partial(
    pl.kernel,
    mesh=_mesh,
    out_type=jax.ShapeDtypeStruct((B, OUT_DIM), jnp.float32),
    compiler_params=pltpu.CompilerParams(
        use_tc_tiling_on_sc=False, needs_layout_passes=False
    ),
    scratch_types=[
        pltpu.VMEM((RPW,), jnp.int32),
        pltpu.VMEM((RPW, EMB), jnp.float32),
        pltpu.VMEM((RPW, N_EMB), jnp.float32),
        pltpu.VMEM((PERW,), jnp.float32),
        pltpu.SemaphoreType.DMA,
    ],
)(_gt_body)


def kernel(x, emb_tables):
    tabt = jnp.transpose(emb_tables, (0, 2, 1)).reshape(N_EMB * EMB, VOCAB)
    packed, idxf, ptf = _repack(x.T, tabt)
    return _gather(idxf, ptf, packed.reshape(NROW, EMB))


# speedup vs baseline: 1.6262x; 1.0049x over previous
import functools

import jax
import jax.numpy as jnp
from jax import lax
from jax.experimental import pallas as pl
from jax.experimental.pallas import tpu as pltpu
from jax.experimental.pallas import tpu_sc as plsc

B = 16384
IN_DIM = 52
N_EMB = 26
VOCAB = 100000
EMB = 16
OUT_DIM = N_EMB * EMB + N_EMB
NROW = N_EMB * VOCAB

NC = 2
NS = 16
L = 16
NW = NC * NS
RPW = B // NW
NCHUNK = RPW // L
IDXW = 128
NIDX = RPW // IDXW
PERW = RPW * N_EMB
HRPW = RPW // 2
HPERW = HRPW * N_EMB

VT = VOCAB // 128
UNITS = N_EMB * VT
NBUF = 4
PADK = 636
TAILV = VOCAB - VT * 128


def _tp_body(x_hbm, tabt_hbm, packed_hbm, idxf_hbm, ptf_hbm,
             xs_v, idxs_v, pts_v, stage, p_v, st_t, p_t,
             sin0, sin1, sin2, sin3, sout0, sout1, sout2, sout3):
    w = lax.axis_index("s") * NC + lax.axis_index("c")
    base = w * RPW

    iota = lax.iota(jnp.int32, L)

    def half_body(cb, c):
        pltpu.sync_copy(x_hbm.at[:, pl.ds(base + cb * HRPW, HRPW)], xs_v)

        @plsc.parallel_loop(0, N_EMB * (HRPW // L), unroll=4)
        def _(ik):
            i = ik // (HRPW // L)
            k = ik % (HRPW // L)
            vals = xs_v[i, pl.ds(k * L, L)] + i * VOCAB
            idxs_v[pl.ds(i * HRPW + k * L, L)] = vals

        @plsc.parallel_loop(0, N_EMB * (HRPW // L), unroll=4)
        def _(jk):
            j = jk // (HRPW // L)
            k = jk % (HRPW // L)
            vals = xs_v[N_EMB + j, pl.ds(k * L, L)].astype(jnp.float32)
            pos = (k * L + iota) * N_EMB + j
            plsc.store_scatter(pts_v, [pos], vals)

        pltpu.sync_copy(idxs_v, idxf_hbm.at[pl.ds(w * PERW + cb * HPERW, HPERW)])
        pltpu.sync_copy(pts_v, ptf_hbm.at[pl.ds(w * PERW + cb * HPERW, HPERW)])
        return c
    lax.fori_loop(0, 2, half_body, 0)

    sins = (sin0, sin1, sin2, sin3)
    souts = (sout0, sout1, sout2, sout3)

    def unit_coords(k):
        u = w + k * NW
        u = lax.select(u < UNITS, u, w)
        i = u // VT
        vt = u % VT
        return i * 16, vt * 128, i * VOCAB + vt * 128

    def issue_load(k, b):
        f0, v0, _ = unit_coords(k)
        pltpu.async_copy(
            tabt_hbm.at[pl.ds(f0, 16), pl.ds(v0, 128)], stage.at[b], sins[b]
        )

    def shuffle(b):
        @plsc.parallel_loop(0, 128, unroll=8)
        def _(j):
            vals = plsc.load_gather(
                stage,
                [jnp.full((L,), b, jnp.int32), iota, jnp.full((L,), j, jnp.int32)],
            )
            p_v[b, pl.ds(j * L, L)] = vals

    for b in range(NBUF):
        issue_load(b, b)

    def pipe_body(kk, c):
        for b in range(NBUF):
            k = kk * NBUF + b
            pltpu.make_async_copy(
                tabt_hbm.at[pl.ds(0, 16), pl.ds(0, 128)], stage.at[b], sins[b]
            ).wait()

            @pl.when(kk > 0)
            def _():
                pltpu.make_async_copy(
                    p_v.at[b], packed_hbm.at[pl.ds(0, 128 * EMB)], souts[b]
                ).wait()

            shuffle(b)
            _, _, q0 = unit_coords(k)
            pltpu.async_copy(
                p_v.at[b], packed_hbm.at[pl.ds(q0 * EMB, 128 * EMB)], souts[b]
            )

            @pl.when(k < PADK - NBUF)
            def _():
                issue_load(k + NBUF, b)
        return c

    lax.fori_loop(0, PADK // NBUF, pipe_body, 0)
    for b in range(NBUF):
        pltpu.make_async_copy(
            p_v.at[b], packed_hbm.at[pl.ds(0, 128 * EMB)], souts[b]
        ).wait()

    @pl.when(w < N_EMB)
    def _():
        pltpu.sync_copy(
            tabt_hbm.at[pl.ds(w * 16, 16), pl.ds(VT * 128, TAILV)], st_t
        )
        @plsc.parallel_loop(0, TAILV, unroll=4)
        def _(j):
            vals = plsc.load_gather(
                st_t, [iota, jnp.full((L,), j, jnp.int32)]
            )
            p_t[pl.ds(j * L, L)] = vals
        pltpu.sync_copy(
            p_t, packed_hbm.at[pl.ds((w * VOCAB + VT * 128) * EMB, TAILV * EMB)]
        )


def _gt_body(idxf_hbm, ptf_hbm, packed_hbm, out_hbm, idx_v, g_v, f_v, pt1_v, sem):
    w = lax.axis_index("s") * NC + lax.axis_index("c")
    base = w * RPW

    pltpu.sync_copy(ptf_hbm.at[pl.ds(w * PERW, PERW)], pt1_v)

    @plsc.parallel_loop(0, RPW, unroll=4)
    def _(r):
        f_v[r, pl.ds(0, L)] = pt1_v[pl.ds(r * N_EMB, L)]
        f_v[r, pl.ds(10, L)] = pt1_v[pl.ds(r * N_EMB + 10, L)]
    pltpu.sync_copy(f_v, out_hbm.at[pl.ds(base, RPW), pl.ds(N_EMB * EMB, N_EMB)])

    def col_body(i, carry):
        pltpu.sync_copy(
            idxf_hbm.at[pl.ds(w * PERW + i * HRPW, HRPW)],
            idx_v.at[pl.ds(0, HRPW)],
        )
        pltpu.sync_copy(
            idxf_hbm.at[pl.ds(w * PERW + HPERW + i * HRPW, HRPW)],
            idx_v.at[pl.ds(HRPW, HRPW)],
        )
        cps = [
            pltpu.async_copy(
                packed_hbm.at[idx_v.at[pl.ds(j * IDXW, IDXW)]],
                g_v.at[pl.ds(j * IDXW, IDXW), :],
                sem,
            )
            for j in range(NIDX)
        ]
        for cp in cps:
            cp.wait()
        pltpu.sync_copy(g_v, out_hbm.at[pl.ds(base, RPW), pl.ds(i * EMB, EMB)])
        return carry

    lax.fori_loop(0, N_EMB, col_body, 0)


_mesh = plsc.VectorSubcoreMesh(core_axis_name="c", subcore_axis_name="s")

_repack = functools.partial(
    pl.kernel,
    mesh=_mesh,
    out_type=(
        jax.ShapeDtypeStruct((NROW * EMB,), jnp.float32),
        jax.ShapeDtypeStruct((B * N_EMB,), jnp.int32),
        jax.ShapeDtypeStruct((B * N_EMB,), jnp.float32),
    ),
    compiler_params=pltpu.CompilerParams(needs_layout_passes=False),
    scratch_types=[
        pltpu.VMEM((IN_DIM, HRPW), jnp.int32),
        pltpu.VMEM((HPERW,), jnp.int32),
        pltpu.VMEM((HPERW,), jnp.float32),
        pltpu.VMEM((NBUF, 16, 128), jnp.float32),
        pltpu.VMEM((NBUF, 128 * EMB), jnp.float32),
        pltpu.VMEM((16, TAILV), jnp.float32),
        pltpu.VMEM((TAILV * EMB,), jnp.float32),
        pltpu.SemaphoreType.DMA,
        pltpu.SemaphoreType.DMA,
        pltpu.SemaphoreType.DMA,
        pltpu.SemaphoreType.DMA,
        pltpu.SemaphoreType.DMA,
        pltpu.SemaphoreType.DMA,
        pltpu.SemaphoreType.DMA,
        pltpu.SemaphoreType.DMA,
    ],
)(_tp_body)

_gather = functools.partial(
    pl.kernel,
    mesh=_mesh,
    out_type=jax.ShapeDtypeStruct((B, OUT_DIM), jnp.float32),
    compiler_params=pltpu.CompilerParams(
        use_tc_tiling_on_sc=False, needs_layout_passes=False
    ),
    scratch_types=[
        pltpu.VMEM((RPW,), jnp.int32),
        pltpu.VMEM((RPW, EMB), jnp.float32),
        pltpu.VMEM((RPW, N_EMB), jnp.float32),
        pltpu.VMEM((PERW,), jnp.float32),
        pltpu.SemaphoreType.DMA,
    ],
)(_gt_body)


def kernel(x, emb_tables):
    tabt = jnp.transpose(emb_tables, (0, 2, 1)).reshape(N_EMB * EMB, VOCAB)
    packed, idxf, ptf = _repack(x.T, tabt)
    return _gather(idxf, ptf, packed.reshape(NROW, EMB))


# speedup vs baseline: 3.1482x; 1.9359x over previous
import functools

import jax
import jax.numpy as jnp
from jax import lax
from jax.experimental import pallas as pl
from jax.experimental.pallas import tpu as pltpu
from jax.experimental.pallas import tpu_sc as plsc

B = 16384
IN_DIM = 52
N_EMB = 26
VOCAB = 100000
EMB = 16
OUT_DIM = N_EMB * EMB + N_EMB
NROW = N_EMB * VOCAB

NC = 2
NS = 16
L = 16
NW = NC * NS
RPW = B // NW
NCHUNK = RPW // L
IDXW = 128
NIDX = RPW // IDXW
PERW = RPW * N_EMB
HRPW = RPW // 2
HPERW = HRPW * N_EMB

VT = VOCAB // 128
UNITS = N_EMB * VT
NBUF = 4
PADK = 636
TAILV = VOCAB - VT * 128


def _tp_body(x_hbm, tabt_hbm, packed_hbm, idxf_hbm, ptf_hbm,
             xs_v, idxs_v, pts_v, stage, p_v, st_t, p_t,
             sin0, sin1, sin2, sin3, sout0, sout1, sout2, sout3):
    w = lax.axis_index("s") * NC + lax.axis_index("c")
    base = w * RPW

    iota = lax.iota(jnp.int32, L)

    def half_body(cb, c):
        pltpu.sync_copy(x_hbm.at[:, pl.ds(base + cb * HRPW, HRPW)], xs_v)

        @plsc.parallel_loop(0, N_EMB * (HRPW // L), unroll=4)
        def _(ik):
            i = ik // (HRPW // L)
            k = ik % (HRPW // L)
            vals = xs_v[i, pl.ds(k * L, L)] + i * VOCAB
            idxs_v[pl.ds(i * HRPW + k * L, L)] = vals

        @plsc.parallel_loop(0, N_EMB * (HRPW // L), unroll=4)
        def _(jk):
            j = jk // (HRPW // L)
            k = jk % (HRPW // L)
            vals = xs_v[N_EMB + j, pl.ds(k * L, L)].astype(jnp.float32)
            pos = (k * L + iota) * N_EMB + j
            plsc.store_scatter(pts_v, [pos], vals)

        pltpu.sync_copy(idxs_v, idxf_hbm.at[pl.ds(w * PERW + cb * HPERW, HPERW)])
        pltpu.sync_copy(pts_v, ptf_hbm.at[pl.ds(w * PERW + cb * HPERW, HPERW)])
        return c
    lax.fori_loop(0, 2, half_body, 0)

    sins = (sin0, sin1, sin2, sin3)
    souts = (sout0, sout1, sout2, sout3)

    def unit_coords(k):
        u = w + k * NW
        u = lax.select(u < UNITS, u, w)
        i = u // VT
        vt = u % VT
        return i * 16, vt * 128, i * VOCAB + vt * 128

    def issue_load(k, b):
        f0, v0, _ = unit_coords(k)
        pltpu.async_copy(
            tabt_hbm.at[pl.ds(f0, 16), pl.ds(v0, 128)], stage.at[b], sins[b]
        )

    def shuffle(b):
        @plsc.parallel_loop(0, 128, unroll=8)
        def _(j):
            blk = (j // 16) * 16
            m = blk + ((j % 16) + iota) % 16
            vals = plsc.load_gather(
                stage, [jnp.full((L,), b, jnp.int32), iota, m]
            )
            plsc.store_scatter(
                p_v, [jnp.full((L,), b, jnp.int32), m * L + iota], vals
            )

    for b in range(NBUF):
        issue_load(b, b)

    def pipe_body(kk, c):
        for b in range(NBUF):
            k = kk * NBUF + b
            pltpu.make_async_copy(
                tabt_hbm.at[pl.ds(0, 16), pl.ds(0, 128)], stage.at[b], sins[b]
            ).wait()

            @pl.when(kk > 0)
            def _():
                pltpu.make_async_copy(
                    p_v.at[b], packed_hbm.at[pl.ds(0, 128 * EMB)], souts[b]
                ).wait()

            shuffle(b)
            _, _, q0 = unit_coords(k)
            pltpu.async_copy(
                p_v.at[b], packed_hbm.at[pl.ds(q0 * EMB, 128 * EMB)], souts[b]
            )

            @pl.when(k < PADK - NBUF)
            def _():
                issue_load(k + NBUF, b)
        return c

    lax.fori_loop(0, PADK // NBUF, pipe_body, 0)
    for b in range(NBUF):
        pltpu.make_async_copy(
            p_v.at[b], packed_hbm.at[pl.ds(0, 128 * EMB)], souts[b]
        ).wait()

    @pl.when(w < N_EMB)
    def _():
        pltpu.sync_copy(
            tabt_hbm.at[pl.ds(w * 16, 16), pl.ds(VT * 128, TAILV)], st_t
        )
        @plsc.parallel_loop(0, TAILV, unroll=4)
        def _(j):
            blk = (j // 16) * 16
            m = blk + ((j % 16) + iota) % 16
            vals = plsc.load_gather(st_t, [iota, m])
            plsc.store_scatter(p_t, [m * L + iota], vals)
        pltpu.sync_copy(
            p_t, packed_hbm.at[pl.ds((w * VOCAB + VT * 128) * EMB, TAILV * EMB)]
        )


def _gt_body(idxf_hbm, ptf_hbm, packed_hbm, out_hbm, idx_v, g_v, f_v, pt1_v, sem):
    w = lax.axis_index("s") * NC + lax.axis_index("c")
    base = w * RPW

    pltpu.sync_copy(ptf_hbm.at[pl.ds(w * PERW, PERW)], pt1_v)

    @plsc.parallel_loop(0, RPW, unroll=4)
    def _(r):
        f_v[r, pl.ds(0, L)] = pt1_v[pl.ds(r * N_EMB, L)]
        f_v[r, pl.ds(10, L)] = pt1_v[pl.ds(r * N_EMB + 10, L)]
    pltpu.sync_copy(f_v, out_hbm.at[pl.ds(base, RPW), pl.ds(N_EMB * EMB, N_EMB)])

    def col_body(i, carry):
        pltpu.sync_copy(
            idxf_hbm.at[pl.ds(w * PERW + i * HRPW, HRPW)],
            idx_v.at[pl.ds(0, HRPW)],
        )
        pltpu.sync_copy(
            idxf_hbm.at[pl.ds(w * PERW + HPERW + i * HRPW, HRPW)],
            idx_v.at[pl.ds(HRPW, HRPW)],
        )
        cps = [
            pltpu.async_copy(
                packed_hbm.at[idx_v.at[pl.ds(j * IDXW, IDXW)]],
                g_v.at[pl.ds(j * IDXW, IDXW), :],
                sem,
            )
            for j in range(NIDX)
        ]
        for cp in cps:
            cp.wait()
        pltpu.sync_copy(g_v, out_hbm.at[pl.ds(base, RPW), pl.ds(i * EMB, EMB)])
        return carry

    lax.fori_loop(0, N_EMB, col_body, 0)


_mesh = plsc.VectorSubcoreMesh(core_axis_name="c", subcore_axis_name="s")

_repack = functools.partial(
    pl.kernel,
    mesh=_mesh,
    out_type=(
        jax.ShapeDtypeStruct((NROW * EMB,), jnp.float32),
        jax.ShapeDtypeStruct((B * N_EMB,), jnp.int32),
        jax.ShapeDtypeStruct((B * N_EMB,), jnp.float32),
    ),
    compiler_params=pltpu.CompilerParams(needs_layout_passes=False),
    scratch_types=[
        pltpu.VMEM((IN_DIM, HRPW), jnp.int32),
        pltpu.VMEM((HPERW,), jnp.int32),
        pltpu.VMEM((HPERW,), jnp.float32),
        pltpu.VMEM((NBUF, 16, 128), jnp.float32),
        pltpu.VMEM((NBUF, 128 * EMB), jnp.float32),
        pltpu.VMEM((16, TAILV), jnp.float32),
        pltpu.VMEM((TAILV * EMB,), jnp.float32),
        pltpu.SemaphoreType.DMA,
        pltpu.SemaphoreType.DMA,
        pltpu.SemaphoreType.DMA,
        pltpu.SemaphoreType.DMA,
        pltpu.SemaphoreType.DMA,
        pltpu.SemaphoreType.DMA,
        pltpu.SemaphoreType.DMA,
        pltpu.SemaphoreType.DMA,
    ],
)(_tp_body)

_gather = functools.partial(
    pl.kernel,
    mesh=_mesh,
    out_type=jax.ShapeDtypeStruct((B, OUT_DIM), jnp.float32),
    compiler_params=pltpu.CompilerParams(
        use_tc_tiling_on_sc=False, needs_layout_passes=False
    ),
    scratch_types=[
        pltpu.VMEM((RPW,), jnp.int32),
        pltpu.VMEM((RPW, EMB), jnp.float32),
        pltpu.VMEM((RPW, N_EMB), jnp.float32),
        pltpu.VMEM((PERW,), jnp.float32),
        pltpu.SemaphoreType.DMA,
    ],
)(_gt_body)


def kernel(x, emb_tables):
    tabt = jnp.transpose(emb_tables, (0, 2, 1)).reshape(N_EMB * EMB, VOCAB)
    packed, idxf, ptf = _repack(x.T, tabt)
    return _gather(idxf, ptf, packed.reshape(NROW, EMB))


# speedup vs baseline: 3.4521x; 1.0965x over previous
import functools

import jax
import jax.numpy as jnp
from jax import lax
from jax.experimental import pallas as pl
from jax.experimental.pallas import tpu as pltpu
from jax.experimental.pallas import tpu_sc as plsc

B = 16384
IN_DIM = 52
N_EMB = 26
VOCAB = 100000
EMB = 16
OUT_DIM = N_EMB * EMB + N_EMB
NROW = N_EMB * VOCAB

NC = 2
NS = 16
L = 16
NW = NC * NS
RPW = B // NW
NCHUNK = RPW // L
IDXW = 128
NIDX = RPW // IDXW
PERW = RPW * N_EMB
HRPW = RPW // 2
HPERW = HRPW * N_EMB

VT = VOCAB // 128
UNITS = N_EMB * VT
NBUF = 4
PADK = 636
TAILV = VOCAB - VT * 128


def _tp_body(x_hbm, tabt_hbm, packed_hbm, idxf_hbm, ptf_hbm,
             xs_v, idxs_v, pts_v, stage, p_v, st_t, p_t,
             sin0, sin1, sin2, sin3, sout0, sout1, sout2, sout3):
    w = lax.axis_index("s") * NC + lax.axis_index("c")
    base = w * RPW

    iota = lax.iota(jnp.int32, L)

    def half_body(cb, c):
        pltpu.sync_copy(x_hbm.at[:, pl.ds(base + cb * HRPW, HRPW)], xs_v)

        @plsc.parallel_loop(0, N_EMB * (HRPW // L), unroll=4)
        def _(ik):
            i = ik // (HRPW // L)
            k = ik % (HRPW // L)
            vals = xs_v[i, pl.ds(k * L, L)] + i * VOCAB
            idxs_v[pl.ds(i * HRPW + k * L, L)] = vals

        @plsc.parallel_loop(0, N_EMB * (HRPW // L), unroll=4)
        def _(jk):
            j = jk // (HRPW // L)
            k = jk % (HRPW // L)
            vals = xs_v[N_EMB + j, pl.ds(k * L, L)].astype(jnp.float32)
            pos = (k * L + iota) * N_EMB + j
            plsc.store_scatter(pts_v, [pos], vals)

        pltpu.sync_copy(idxs_v, idxf_hbm.at[pl.ds(w * PERW + cb * HPERW, HPERW)])
        pltpu.sync_copy(pts_v, ptf_hbm.at[pl.ds(w * PERW + cb * HPERW, HPERW)])
        return c
    lax.fori_loop(0, 2, half_body, 0)

    sins = (sin0, sin1, sin2, sin3)
    souts = (sout0, sout1, sout2, sout3)

    def unit_coords(k):
        u = w + k * NW
        u = lax.select(u < UNITS, u, w)
        i = u // VT
        vt = u % VT
        return i * 16, vt * 128, i * VOCAB + vt * 128

    def issue_load(k, b):
        f0, v0, _ = unit_coords(k)
        pltpu.async_copy(
            tabt_hbm.at[pl.ds(f0, 16), pl.ds(v0, 128)], stage.at[b], sins[b]
        )

    def shuffle(b):
        @plsc.parallel_loop(0, 128, unroll=8)
        def _(j):
            blk = (j // 16) * 16
            m = blk + ((j % 16) + iota) % 16
            vals = plsc.load_gather(
                stage, [jnp.full((L,), b, jnp.int32), iota, m]
            )
            plsc.store_scatter(
                p_v, [jnp.full((L,), b, jnp.int32), m * L + iota], vals
            )

    for b in range(NBUF):
        issue_load(b, b)

    def pipe_body(kk, c):
        for b in range(NBUF):
            k = kk * NBUF + b
            pltpu.make_async_copy(
                tabt_hbm.at[pl.ds(0, 16), pl.ds(0, 128)], stage.at[b], sins[b]
            ).wait()

            @pl.when(kk > 0)
            def _():
                pltpu.make_async_copy(
                    p_v.at[b], packed_hbm.at[pl.ds(0, 128 * EMB)], souts[b]
                ).wait()

            shuffle(b)
            _, _, q0 = unit_coords(k)
            pltpu.async_copy(
                p_v.at[b], packed_hbm.at[pl.ds(q0 * EMB, 128 * EMB)], souts[b]
            )

            @pl.when(k < PADK - NBUF)
            def _():
                issue_load(k + NBUF, b)
        return c

    lax.fori_loop(0, PADK // NBUF, pipe_body, 0)
    for b in range(NBUF):
        pltpu.make_async_copy(
            p_v.at[b], packed_hbm.at[pl.ds(0, 128 * EMB)], souts[b]
        ).wait()

    @pl.when(w < N_EMB)
    def _():
        pltpu.sync_copy(
            tabt_hbm.at[pl.ds(w * 16, 16), pl.ds(VT * 128, TAILV)], st_t
        )
        @plsc.parallel_loop(0, TAILV, unroll=4)
        def _(j):
            blk = (j // 16) * 16
            m = blk + ((j % 16) + iota) % 16
            vals = plsc.load_gather(st_t, [iota, m])
            plsc.store_scatter(p_t, [m * L + iota], vals)
        pltpu.sync_copy(
            p_t, packed_hbm.at[pl.ds((w * VOCAB + VT * 128) * EMB, TAILV * EMB)]
        )


def _gt_body(idxf_hbm, ptf_hbm, packed_hbm, out_hbm,
             idx_v, g_v, f_v, pt1_v, si0, si1, sg0, sg1, sw0, sw1):
    w = lax.axis_index("s") * NC + lax.axis_index("c")
    base = w * RPW
    sis = (si0, si1)
    sgs = (sg0, sg1)
    sws = (sw0, sw1)

    pltpu.sync_copy(ptf_hbm.at[pl.ds(w * PERW, PERW)], pt1_v)

    @plsc.parallel_loop(0, RPW, unroll=4)
    def _(r):
        f_v[r, pl.ds(0, L)] = pt1_v[pl.ds(r * N_EMB, L)]
        f_v[r, pl.ds(10, L)] = pt1_v[pl.ds(r * N_EMB + 10, L)]
    pltpu.sync_copy(f_v, out_hbm.at[pl.ds(base, RPW), pl.ds(N_EMB * EMB, N_EMB)])

    def load_idx(i, b, s):
        src0 = idxf_hbm.at[pl.ds(w * PERW + i * HRPW, HRPW)]
        src1 = idxf_hbm.at[pl.ds(w * PERW + HPERW + i * HRPW, HRPW)]
        if s:
            pltpu.sync_copy(src0, idx_v.at[b, pl.ds(0, HRPW)])
            pltpu.sync_copy(src1, idx_v.at[b, pl.ds(HRPW, HRPW)])
        else:
            pltpu.async_copy(src0, idx_v.at[b, pl.ds(0, HRPW)], sis[b])
            pltpu.async_copy(src1, idx_v.at[b, pl.ds(HRPW, HRPW)], sis[b])

    def wait_idx(b):
        for h in range(2):
            pltpu.make_async_copy(
                idxf_hbm.at[pl.ds(0, HRPW)], idx_v.at[b, pl.ds(0, HRPW)], sis[b]
            ).wait()

    def issue_gathers(b):
        for j in range(NIDX):
            pltpu.async_copy(
                packed_hbm.at[idx_v.at[b, pl.ds(j * IDXW, IDXW)]],
                g_v.at[b, pl.ds(j * IDXW, IDXW), :],
                sgs[b],
            )

    def wait_gathers(b):
        for j in range(NIDX):
            pltpu.make_async_copy(
                packed_hbm.at[idx_v.at[b, pl.ds(0, IDXW)]],
                g_v.at[b, pl.ds(0, IDXW), :],
                sgs[b],
            ).wait()

    def wait_write(b):
        pltpu.make_async_copy(
            g_v.at[b], out_hbm.at[pl.ds(base, RPW), pl.ds(0, EMB)], sws[b]
        ).wait()

    load_idx(0, 0, True)
    issue_gathers(0)
    load_idx(1, 1, False)

    def col_body(ii, carry):
        for b in range(2):
            i = ii * 2 + b
            nb = 1 - b
            wait_gathers(b)
            pltpu.async_copy(
                g_v.at[b], out_hbm.at[pl.ds(base, RPW), pl.ds(i * EMB, EMB)], sws[b]
            )

            @pl.when(i + 1 < N_EMB)
            def _():
                wait_idx(nb)

                @pl.when(i >= 1)
                def _():
                    wait_write(nb)
                issue_gathers(nb)

            @pl.when(i + 2 < N_EMB)
            def _():
                load_idx(i + 2, b, False)
        return carry

    lax.fori_loop(0, N_EMB // 2, col_body, 0)
    wait_write(0)
    wait_write(1)


_mesh = plsc.VectorSubcoreMesh(core_axis_name="c", subcore_axis_name="s")

_repack = functools.partial(
    pl.kernel,
    mesh=_mesh,
    out_type=(
        jax.ShapeDtypeStruct((NROW * EMB,), jnp.float32),
        jax.ShapeDtypeStruct((B * N_EMB,), jnp.int32),
        jax.ShapeDtypeStruct((B * N_EMB,), jnp.float32),
    ),
    compiler_params=pltpu.CompilerParams(needs_layout_passes=False),
    scratch_types=[
        pltpu.VMEM((IN_DIM, HRPW), jnp.int32),
        pltpu.VMEM((HPERW,), jnp.int32),
        pltpu.VMEM((HPERW,), jnp.float32),
        pltpu.VMEM((NBUF, 16, 128), jnp.float32),
        pltpu.VMEM((NBUF, 128 * EMB), jnp.float32),
        pltpu.VMEM((16, TAILV), jnp.float32),
        pltpu.VMEM((TAILV * EMB,), jnp.float32),
        pltpu.SemaphoreType.DMA,
        pltpu.SemaphoreType.DMA,
        pltpu.SemaphoreType.DMA,
        pltpu.SemaphoreType.DMA,
        pltpu.SemaphoreType.DMA,
        pltpu.SemaphoreType.DMA,
        pltpu.SemaphoreType.DMA,
        pltpu.SemaphoreType.DMA,
    ],
)(_tp_body)

_gather = functools.partial(
    pl.kernel,
    mesh=_mesh,
    out_type=jax.ShapeDtypeStruct((B, OUT_DIM), jnp.float32),
    compiler_params=pltpu.CompilerParams(
        use_tc_tiling_on_sc=False, needs_layout_passes=False
    ),
    scratch_types=[
        pltpu.VMEM((2, RPW), jnp.int32),
        pltpu.VMEM((2, RPW, EMB), jnp.float32),
        pltpu.VMEM((RPW, N_EMB), jnp.float32),
        pltpu.VMEM((PERW,), jnp.float32),
        pltpu.SemaphoreType.DMA,
        pltpu.SemaphoreType.DMA,
        pltpu.SemaphoreType.DMA,
        pltpu.SemaphoreType.DMA,
        pltpu.SemaphoreType.DMA,
        pltpu.SemaphoreType.DMA,
    ],
)(_gt_body)


def kernel(x, emb_tables):
    tabt = jnp.transpose(emb_tables, (0, 2, 1)).reshape(N_EMB * EMB, VOCAB)
    packed, idxf, ptf = _repack(x.T, tabt)
    return _gather(idxf, ptf, packed.reshape(NROW, EMB))
